# baseline (speedup 1.0000x reference)
import functools

import jax
import jax.numpy as jnp
from jax import lax
from jax.experimental import pallas as pl
from jax.experimental.pallas import tpu as pltpu
from jax.experimental.pallas import tpu_sc as plsc

N_EMB = 64
BATCH = 16384
NC, NS, L = 2, 16, 16
NW = NC * NS
BPW = BATCH // NW
CHUNK = 128
NCH = BPW // CHUNK
NVR = N_EMB // L
PROWS = 500000


def _gmf_body(users_hbm, items_hbm, utab_hbm, itab_hbm, par_hbm, out_hbm,
              uidx_v, iidx_v, uphys_v, iphys_v, upar_v, ipar_v,
              urows0, urows1, irows0, irows1, par_v, wsplat_v, out_v, *sems):
    wid = lax.axis_index("s") * NC + lax.axis_index("c")
    base = wid * BPW

    pltpu.sync_copy(users_hbm.at[wid], uidx_v)
    pltpu.sync_copy(items_hbm.at[wid], iidx_v)
    pltpu.sync_copy(par_hbm, par_v)

    for c in range(NCH):
        for j in range(CHUNK // L):
            u = uidx_v[c, pl.ds(j * L, L)]
            v = iidx_v[c, pl.ds(j * L, L)]
            uphys_v[c, pl.ds(j * L, L)] = u >> 1
            iphys_v[c, pl.ds(j * L, L)] = v >> 1
            upar_v[pl.ds(c * CHUNK + j * L, L)] = (u & 1) * N_EMB
            ipar_v[pl.ds(c * CHUNK + j * L, L)] = (v & 1) * N_EMB

    urows = (urows0, urows1)
    irows = (irows0, irows1)

    def fire(c):
        s = c % 2
        cu = pltpu.async_copy(utab_hbm.at[uphys_v.at[c]], urows[s], sems[2 * s])
        ci = pltpu.async_copy(itab_hbm.at[iphys_v.at[c]], irows[s], sems[2 * s + 1])
        return (cu, ci)

    copies = [fire(0), fire(1)]

    for d in range(N_EMB):
        wsplat_v[d, :] = plsc.load_gather(par_v, [jnp.full((L,), d, jnp.int32)])
    bias = par_v[pl.ds(N_EMB, L)]
    lane = lax.iota(jnp.int32, L)

    for c in range(NCH):
        copies[c % 2][0].wait()
        copies[c % 2][1].wait()
        ub, ib = urows[c % 2], irows[c % 2]

        def body(g, _, c=c, ub=ub, ib=ib):
            row0 = g * L
            rows = row0 + lane
            ucols = upar_v[pl.ds(c * CHUNK + row0, L)]
            icols = ipar_v[pl.ds(c * CHUNK + row0, L)]
            acc = jnp.zeros((L,), jnp.float32)
            for d in range(N_EMB):
                cu = plsc.load_gather(ub, [rows, ucols + d])
                ci = plsc.load_gather(ib, [rows, icols + d])
                acc = acc + (cu * ci) * wsplat_v[d, :]
            z = acc + bias
            p = 1.0 / (1.0 + jnp.exp(-z))
            out_v[pl.ds(c * CHUNK + row0, L)] = p
            return _

        lax.fori_loop(0, CHUNK // L, body, 0)
        if c + 2 < NCH:
            copies[c % 2] = fire(c + 2)

    pltpu.sync_copy(out_v, out_hbm.at[pl.ds(base, BPW)])


@jax.jit
def _gmf(users3, items3, utab2, itab2, par):
    mesh = plsc.VectorSubcoreMesh(core_axis_name="c", subcore_axis_name="s",
                                  num_cores=NC, num_subcores=NS)
    scratch = [
        pltpu.VMEM((NCH, CHUNK), jnp.int32),
        pltpu.VMEM((NCH, CHUNK), jnp.int32),
        pltpu.VMEM((NCH, CHUNK), jnp.int32),
        pltpu.VMEM((NCH, CHUNK), jnp.int32),
        pltpu.VMEM((BPW,), jnp.int32),
        pltpu.VMEM((BPW,), jnp.int32),
        pltpu.VMEM((CHUNK, 2 * N_EMB), jnp.float32),
        pltpu.VMEM((CHUNK, 2 * N_EMB), jnp.float32),
        pltpu.VMEM((CHUNK, 2 * N_EMB), jnp.float32),
        pltpu.VMEM((CHUNK, 2 * N_EMB), jnp.float32),
        pltpu.VMEM((5 * L,), jnp.float32),
        pltpu.VMEM((N_EMB, L), jnp.float32),
        pltpu.VMEM((BPW,), jnp.float32),
    ] + [pltpu.SemaphoreType.DMA] * 4
    run = pl.kernel(
        _gmf_body,
        out_type=jax.ShapeDtypeStruct((BATCH,), jnp.float32),
        mesh=mesh,
        scratch_types=scratch,
        compiler_params=pltpu.CompilerParams(needs_layout_passes=False,
                                             use_tc_tiling_on_sc=True),
    )
    return run(users3, items3, utab2, itab2, par)


def kernel(users, items, user_table, item_table, W, b):
    users3 = users.reshape(NW, NCH, CHUNK)
    items3 = items.reshape(NW, NCH, CHUNK)
    utab2 = user_table.reshape(PROWS, 2 * N_EMB)
    itab2 = item_table.reshape(PROWS, 2 * N_EMB)
    par = jnp.concatenate(
        [W.reshape(-1), jnp.full((L,), b[0], jnp.float32)])
    out = _gmf(users3, items3, utab2, itab2, par)
    return out.reshape(BATCH, 1)
